# baseline (speedup 1.0000x reference)
import functools

import jax
import jax.numpy as jnp
from jax import lax
from jax.experimental import pallas as pl
from jax.experimental.pallas import tpu as pltpu
from jax.experimental.pallas import tpu_sc as plsc

N = 10000
E = 320000
D = 128
H = 128
K = 4
C = 16

_NC = 2
_NS = 16
_NW = _NC * _NS
_HH = H // _NC
_CH = 125
_EPT = E // _NW
_NCH = _EPT // _CH
_EPT2 = E // _NS
_NCH2 = _EPT2 // _CH
_RPT = N // _NS


@functools.lru_cache(maxsize=None)
def _sc_degree_kernel():
    mesh = plsc.VectorSubcoreMesh(core_axis_name="c", subcore_axis_name="s")
    return functools.partial(
        pl.kernel,
        out_type=jax.ShapeDtypeStruct((_NC, N, 16), jnp.float32),
        mesh=mesh,
        scratch_types=[
            pltpu.VMEM((_NCH, _CH), jnp.int32),
            pltpu.VMEM((_CH, 16), jnp.float32),
            pltpu.VMEM_SHARED((N, 16), jnp.float32),
            pltpu.SemaphoreType.DMA,
        ],
        compiler_params=pltpu.CompilerParams(use_tc_tiling_on_sc=False),
    )(_sc_degree_body)


def _sc_degree(col3, ones16, z16):
    return _sc_degree_kernel()(col3, ones16, z16)


def _sc_degree_body(col_hbm, ones_hbm, zrows_hbm, out_hbm, colv, onesv, acc, sem):
    cid = lax.axis_index("c")
    sid = lax.axis_index("s")
    wid = cid * _NS + sid
    pltpu.sync_copy(col_hbm.at[wid], colv)
    pltpu.sync_copy(ones_hbm, onesv)
    pltpu.sync_copy(zrows_hbm, acc.at[pl.ds(sid * _RPT, _RPT)])
    plsc.subcore_barrier()

    def body(j, carry):
        pltpu.sync_copy(onesv, acc.at[colv.at[j]], add=True)
        return carry

    lax.fori_loop(0, _NCH, body, 0)
    plsc.subcore_barrier()
    pltpu.sync_copy(acc.at[pl.ds(sid * _RPT, _RPT)],
                    out_hbm.at[cid, pl.ds(sid * _RPT, _RPT)])


@functools.lru_cache(maxsize=None)
def _sc_conv_kernel():
    mesh = plsc.VectorSubcoreMesh(core_axis_name="c", subcore_axis_name="s")
    return functools.partial(
        pl.kernel,
        out_type=jax.ShapeDtypeStruct((_NC, N, _HH), jnp.float32),
        mesh=mesh,
        scratch_types=[
            pltpu.VMEM((_NCH2, _CH), jnp.int32),
            pltpu.VMEM((_NCH2, _CH), jnp.int32),
            pltpu.VMEM((4, _CH, _HH), jnp.float32),
            pltpu.VMEM_SHARED((N, _HH), jnp.float32),
            [pltpu.SemaphoreType.DMA] * 4,
            [pltpu.SemaphoreType.DMA] * 4,
        ],
        compiler_params=pltpu.CompilerParams(use_tc_tiling_on_sc=False),
    )(_sc_conv_body)


def _sc_conv(xs2, row3, col3, zH):
    return _sc_conv_kernel()(xs2, row3, col3, zH)


def _sc_conv_body(xs_hbm, row_hbm, col_hbm, zrows_hbm, out_hbm,
                  rowv, colv, gbuf, acc, gsems, ssems):
    cid = lax.axis_index("c")
    sid = lax.axis_index("s")
    table = xs_hbm.at[cid]
    pltpu.sync_copy(row_hbm.at[sid], rowv)
    pltpu.sync_copy(col_hbm.at[sid], colv)
    pltpu.sync_copy(zrows_hbm, acc.at[pl.ds(sid * _RPT, _RPT)])
    plsc.subcore_barrier()

    pltpu.async_copy(table.at[rowv.at[0]], gbuf.at[0], gsems[0])
    pltpu.async_copy(table.at[rowv.at[1]], gbuf.at[1], gsems[1])

    def body(mm, carry):
        for u in range(4):
            m = mm * 4 + u
            b = u
            c = (u + 2) % 4
            pltpu.make_async_copy(table.at[rowv.at[m]], gbuf.at[b],
                                  gsems[b]).wait()
            pltpu.async_copy(gbuf.at[b], acc.at[colv.at[m]], ssems[b],
                             add=True)

            @pl.when(m >= 2)
            def _():
                pltpu.make_async_copy(gbuf.at[c], acc.at[colv.at[m - 2]],
                                      ssems[c]).wait()

            @pl.when(m + 2 < _NCH2)
            def _():
                pltpu.async_copy(table.at[rowv.at[m + 2]], gbuf.at[c],
                                 gsems[c])
        return carry

    lax.fori_loop(0, _NCH2 // 4, body, 0)
    pltpu.make_async_copy(gbuf.at[2], acc.at[colv.at[_NCH2 - 2]],
                          ssems[2]).wait()
    pltpu.make_async_copy(gbuf.at[3], acc.at[colv.at[_NCH2 - 1]],
                          ssems[3]).wait()
    plsc.subcore_barrier()
    pltpu.sync_copy(acc.at[pl.ds(sid * _RPT, _RPT)],
                    out_hbm.at[cid, pl.ds(sid * _RPT, _RPT)])


_BLK = 1000
_GRID = N // _BLK


def _dis_from_parts(dp):
    deg = dp[0, :, 0:1] + dp[1, :, 0:1]
    return jnp.where(deg > 0, lax.rsqrt(deg), 0.0)


def _split_store(xs_ref, xs):
    xs_ref[0] = xs[:, :_HH]
    xs_ref[1] = xs[:, _HH:]


def _tc_pre_body(x_ref, w_ref, b_ref, dp_ref, h_ref, xs_ref):
    h = jnp.maximum(x_ref[...] @ w_ref[...] + b_ref[...], 0.0)
    dis = _dis_from_parts(dp_ref[...])
    h_ref[...] = h
    _split_store(xs_ref, h * dis)


def _mix(h, agg, dp, ewp, ebp, wa, wb, s):
    dis = _dis_from_parts(dp)
    hi = jnp.concatenate([agg[0], agg[1]], axis=1) * dis
    logits = h @ ewp + ebp
    m = jnp.max(logits, axis=-1, keepdims=True)
    p = jnp.exp(logits - m)
    e = p / jnp.sum(p, axis=-1, keepdims=True)
    mm = hi @ wa + h @ wb
    ew = e @ s
    pr = mm * ew
    out = pr[:, 0:128] + pr[:, 128:256] + pr[:, 256:384] + pr[:, 384:512] + h
    return jnp.maximum(out, 0.0), dis


def _tc_layer_body(h_ref, agg_ref, dp_ref, ewp_ref, ebp_ref, wa_ref, wb_ref,
                   s_ref, hn_ref, xs_ref):
    hn, dis = _mix(h_ref[...], agg_ref[...], dp_ref[...], ewp_ref[...],
                   ebp_ref[...], wa_ref[...], wb_ref[...], s_ref[...])
    hn_ref[...] = hn
    _split_store(xs_ref, hn * dis)


def _tc_final_body(h_ref, agg_ref, dp_ref, ewp_ref, ebp_ref, wa_ref, wb_ref,
                   s_ref, wo_ref, bo_ref, out_ref):
    hn, _ = _mix(h_ref[...], agg_ref[...], dp_ref[...], ewp_ref[...],
                 ebp_ref[...], wa_ref[...], wb_ref[...], s_ref[...])
    out_ref[...] = hn @ wo_ref[...] + bo_ref[...]


_row_spec = pl.BlockSpec((_BLK, H), lambda i: (i, 0))
_dp_spec = pl.BlockSpec((2, _BLK, 16), lambda i: (0, i, 0))
_agg_spec = pl.BlockSpec((2, _BLK, _HH), lambda i: (0, i, 0))
_xs_shape = jax.ShapeDtypeStruct((_NC, N, _HH), jnp.float32)
_w_spec = pl.BlockSpec((H, H), lambda i: (0, 0))
_b_spec = pl.BlockSpec((1, H), lambda i: (0, 0))
_wcat_spec = pl.BlockSpec((H, K * H), lambda i: (0, 0))


def _tc_pre(x, w, b2, dp):
    return pl.pallas_call(
        _tc_pre_body,
        grid=(_GRID,),
        in_specs=[_row_spec, _w_spec, _b_spec, _dp_spec],
        out_specs=[_row_spec, _agg_spec],
        out_shape=[jax.ShapeDtypeStruct((N, H), jnp.float32), _xs_shape],
    )(x, w, b2, dp)


def _tc_layer(h, agg, dp, ewp, ebp, wa, wb, s):
    return pl.pallas_call(
        _tc_layer_body,
        grid=(_GRID,),
        in_specs=[_row_spec, _agg_spec, _dp_spec, _w_spec, _b_spec,
                  _wcat_spec, _wcat_spec, _wcat_spec],
        out_specs=[_row_spec, _agg_spec],
        out_shape=[jax.ShapeDtypeStruct((N, H), jnp.float32), _xs_shape],
    )(h, agg, dp, ewp, ebp, wa, wb, s)


def _tc_final(h, agg, dp, ewp, ebp, wa, wb, s, wo, bo):
    return pl.pallas_call(
        _tc_final_body,
        grid=(_GRID,),
        in_specs=[_row_spec, _agg_spec, _dp_spec, _w_spec, _b_spec,
                  _wcat_spec, _wcat_spec, _wcat_spec, _w_spec, _b_spec],
        out_specs=_row_spec,
        out_shape=jax.ShapeDtypeStruct((N, H), jnp.float32),
    )(h, agg, dp, ewp, ebp, wa, wb, s, wo, bo)


def _pad_env(env_W, env_b):
    ewp = jnp.zeros((H, H), jnp.float32).at[:, :K].set(env_W)
    ebp = jnp.full((1, H), -1e30, jnp.float32).at[0, :K].set(env_b)
    return ewp, ebp


def kernel(x, edge_index, W_in, b_in, env_W1, env_b1, conv_W1,
           env_W2, env_b2, conv_W2, W_out, b_out):
    rowc = edge_index[0].reshape(_NS, _NCH2, _CH)
    colc = edge_index[1].reshape(_NS, _NCH2, _CH)
    cold = edge_index[1].reshape(_NW, _NCH, _CH)
    ones16 = jnp.ones((_CH, 16), jnp.float32)
    z16 = jnp.zeros((_RPT, 16), jnp.float32)
    zH = jnp.zeros((_RPT, _HH), jnp.float32)

    dp = _sc_degree(cold, ones16, z16)

    ewp1, ebp1 = _pad_env(env_W1, env_b1)
    ewp2, ebp2 = _pad_env(env_W2, env_b2)
    wa1 = jnp.transpose(conv_W1[:, :H, :], (1, 0, 2)).reshape(H, K * H)
    wb1 = jnp.transpose(conv_W1[:, H:, :], (1, 0, 2)).reshape(H, K * H)
    wa2 = jnp.transpose(conv_W2[:, :H, :], (1, 0, 2)).reshape(H, K * H)
    wb2 = jnp.transpose(conv_W2[:, H:, :], (1, 0, 2)).reshape(H, K * H)
    sel = jnp.concatenate(
        [jnp.kron(jnp.eye(K, dtype=jnp.float32), jnp.ones((1, H), jnp.float32)),
         jnp.zeros((H - K, K * H), jnp.float32)], axis=0)
    wo = jnp.zeros((H, H), jnp.float32).at[:, :C].set(W_out)
    bo = jnp.zeros((1, H), jnp.float32).at[0, :C].set(b_out)

    h1, xs1 = _tc_pre(x, W_in, b_in.reshape(1, H), dp)
    agg1 = _sc_conv(xs1, rowc, colc, zH)
    h2, xs2 = _tc_layer(h1, agg1, dp, ewp1, ebp1, wa1, wb1, sel)
    agg2 = _sc_conv(xs2, rowc, colc, zH)
    out_pad = _tc_final(h2, agg2, dp, ewp2, ebp2, wa2, wb2, sel, wo, bo)
    return out_pad[:, :C]


# speedup vs baseline: 1.0116x; 1.0116x over previous
import functools

import jax
import jax.numpy as jnp
from jax import lax
from jax.experimental import pallas as pl
from jax.experimental.pallas import tpu as pltpu
from jax.experimental.pallas import tpu_sc as plsc

N = 10000
E = 320000
D = 128
H = 128
K = 4
C = 16

_NC = 2
_NS = 16
_NW = _NC * _NS
_EPT = E // _NW
_CH = 125
_NCH = _EPT // _CH
_RPT = N // _NS


@functools.lru_cache(maxsize=None)
def _sc_degree_kernel():
    mesh = plsc.VectorSubcoreMesh(core_axis_name="c", subcore_axis_name="s")
    return functools.partial(
        pl.kernel,
        out_type=jax.ShapeDtypeStruct((_NC, N, 16), jnp.float32),
        mesh=mesh,
        scratch_types=[
            pltpu.VMEM((_NCH, _CH), jnp.int32),
            pltpu.VMEM((_CH, 16), jnp.float32),
            pltpu.VMEM_SHARED((N, 16), jnp.float32),
            pltpu.SemaphoreType.DMA,
        ],
        compiler_params=pltpu.CompilerParams(use_tc_tiling_on_sc=False),
    )(_sc_degree_body)


def _sc_degree(col3, ones16, z16):
    return _sc_degree_kernel()(col3, ones16, z16)


def _sc_degree_body(col_hbm, ones_hbm, zrows_hbm, out_hbm, colv, onesv, acc, sem):
    cid = lax.axis_index("c")
    sid = lax.axis_index("s")
    wid = cid * _NS + sid
    pltpu.sync_copy(col_hbm.at[wid], colv)
    pltpu.sync_copy(ones_hbm, onesv)
    pltpu.sync_copy(zrows_hbm, acc.at[pl.ds(sid * _RPT, _RPT)])
    plsc.subcore_barrier()

    def body(j, carry):
        pltpu.sync_copy(onesv, acc.at[colv.at[j]], add=True)
        return carry

    lax.fori_loop(0, _NCH, body, 0)
    plsc.subcore_barrier()
    pltpu.sync_copy(acc.at[pl.ds(sid * _RPT, _RPT)],
                    out_hbm.at[cid, pl.ds(sid * _RPT, _RPT)])


@functools.lru_cache(maxsize=None)
def _sc_conv_kernel():
    mesh = plsc.VectorSubcoreMesh(core_axis_name="c", subcore_axis_name="s")
    return functools.partial(
        pl.kernel,
        out_type=jax.ShapeDtypeStruct((_NC, N, H), jnp.float32),
        mesh=mesh,
        scratch_types=[
            pltpu.VMEM((_NCH // 2, _CH), jnp.int32),
            pltpu.VMEM((_NCH // 2, _CH), jnp.int32),
            pltpu.VMEM((2, _CH, H), jnp.float32),
            pltpu.VMEM_SHARED((N, H), jnp.float32),
            pltpu.SemaphoreType.DMA,
            pltpu.SemaphoreType.DMA,
            pltpu.SemaphoreType.DMA,
            pltpu.SemaphoreType.DMA,
        ],
        compiler_params=pltpu.CompilerParams(use_tc_tiling_on_sc=False),
    )(_sc_conv_body)


def _sc_conv(xs, row3, col3, zH):
    return _sc_conv_kernel()(xs, row3, col3, zH)


def _sc_conv_body(xs_hbm, row_hbm, col_hbm, zrows_hbm, out_hbm,
                  rowv, colv, gbuf, acc, gsem0, gsem1, ssem0, ssem1):
    cid = lax.axis_index("c")
    sid = lax.axis_index("s")
    wid = cid * _NS + sid
    nh = _NCH // 2
    pltpu.sync_copy(zrows_hbm, acc.at[pl.ds(sid * _RPT, _RPT)])
    plsc.subcore_barrier()

    gsems = (gsem0, gsem1)
    ssems = (ssem0, ssem1)
    for half in range(2):
        pltpu.sync_copy(row_hbm.at[wid, pl.ds(half * nh, nh)], rowv)
        pltpu.sync_copy(col_hbm.at[wid, pl.ds(half * nh, nh)], colv)
        pltpu.async_copy(xs_hbm.at[rowv.at[0]], gbuf.at[0], gsem0)

        def body(jj, carry):
            for b in range(2):
                j = jj * 2 + b
                pltpu.make_async_copy(xs_hbm.at[rowv.at[j]], gbuf.at[b],
                                      gsems[b]).wait()
                pltpu.async_copy(gbuf.at[b], acc.at[colv.at[j]], ssems[b],
                                 add=True)

                @pl.when(j >= 1)
                def _():
                    pltpu.make_async_copy(gbuf.at[1 - b],
                                          acc.at[colv.at[j - 1]],
                                          ssems[1 - b]).wait()

                @pl.when(j + 1 < nh)
                def _():
                    pltpu.async_copy(xs_hbm.at[rowv.at[j + 1]],
                                     gbuf.at[1 - b], gsems[1 - b])
            return carry

        lax.fori_loop(0, nh // 2, body, 0)
        pltpu.make_async_copy(gbuf.at[1], acc.at[colv.at[nh - 1]],
                              ssems[1]).wait()
    plsc.subcore_barrier()
    pltpu.sync_copy(acc.at[pl.ds(sid * _RPT, _RPT)],
                    out_hbm.at[cid, pl.ds(sid * _RPT, _RPT)])


_BLK = 1000
_GRID = N // _BLK


def _dis_from_parts(dp):
    deg = dp[0, :, 0:1] + dp[1, :, 0:1]
    return jnp.where(deg > 0, lax.rsqrt(deg), 0.0)


def _tc_pre_body(x_ref, w_ref, b_ref, dp_ref, h_ref, xs_ref):
    h = jnp.maximum(x_ref[...] @ w_ref[...] + b_ref[...], 0.0)
    dis = _dis_from_parts(dp_ref[...])
    h_ref[...] = h
    xs_ref[...] = h * dis


def _mix(h, agg, dp, ewp, ebp, wa, wb, s):
    dis = _dis_from_parts(dp)
    hi = (agg[0] + agg[1]) * dis
    logits = h @ ewp + ebp
    m = jnp.max(logits, axis=-1, keepdims=True)
    p = jnp.exp(logits - m)
    e = p / jnp.sum(p, axis=-1, keepdims=True)
    mm = hi @ wa + h @ wb
    ew = e @ s
    pr = mm * ew
    out = pr[:, 0:128] + pr[:, 128:256] + pr[:, 256:384] + pr[:, 384:512] + h
    return jnp.maximum(out, 0.0), dis


def _tc_layer_body(h_ref, agg_ref, dp_ref, ewp_ref, ebp_ref, wa_ref, wb_ref,
                   s_ref, hn_ref, xs_ref):
    hn, dis = _mix(h_ref[...], agg_ref[...], dp_ref[...], ewp_ref[...],
                   ebp_ref[...], wa_ref[...], wb_ref[...], s_ref[...])
    hn_ref[...] = hn
    xs_ref[...] = hn * dis


def _tc_final_body(h_ref, agg_ref, dp_ref, ewp_ref, ebp_ref, wa_ref, wb_ref,
                   s_ref, wo_ref, bo_ref, out_ref):
    hn, _ = _mix(h_ref[...], agg_ref[...], dp_ref[...], ewp_ref[...],
                 ebp_ref[...], wa_ref[...], wb_ref[...], s_ref[...])
    out_ref[...] = hn @ wo_ref[...] + bo_ref[...]


_row_spec = pl.BlockSpec((_BLK, H), lambda i: (i, 0))
_dp_spec = pl.BlockSpec((2, _BLK, 16), lambda i: (0, i, 0))
_agg_spec = pl.BlockSpec((2, _BLK, H), lambda i: (0, i, 0))
_w_spec = pl.BlockSpec((H, H), lambda i: (0, 0))
_b_spec = pl.BlockSpec((1, H), lambda i: (0, 0))
_wcat_spec = pl.BlockSpec((H, K * H), lambda i: (0, 0))


def _tc_pre(x, w, b2, dp):
    return pl.pallas_call(
        _tc_pre_body,
        grid=(_GRID,),
        in_specs=[_row_spec, _w_spec, _b_spec, _dp_spec],
        out_specs=[_row_spec, _row_spec],
        out_shape=[jax.ShapeDtypeStruct((N, H), jnp.float32)] * 2,
    )(x, w, b2, dp)


def _tc_layer(h, agg, dp, ewp, ebp, wa, wb, s):
    return pl.pallas_call(
        _tc_layer_body,
        grid=(_GRID,),
        in_specs=[_row_spec, _agg_spec, _dp_spec, _w_spec, _b_spec,
                  _wcat_spec, _wcat_spec, _wcat_spec],
        out_specs=[_row_spec, _row_spec],
        out_shape=[jax.ShapeDtypeStruct((N, H), jnp.float32)] * 2,
    )(h, agg, dp, ewp, ebp, wa, wb, s)


def _tc_final(h, agg, dp, ewp, ebp, wa, wb, s, wo, bo):
    return pl.pallas_call(
        _tc_final_body,
        grid=(_GRID,),
        in_specs=[_row_spec, _agg_spec, _dp_spec, _w_spec, _b_spec,
                  _wcat_spec, _wcat_spec, _wcat_spec, _w_spec, _b_spec],
        out_specs=_row_spec,
        out_shape=jax.ShapeDtypeStruct((N, H), jnp.float32),
    )(h, agg, dp, ewp, ebp, wa, wb, s, wo, bo)


def _pad_env(env_W, env_b):
    ewp = jnp.zeros((H, H), jnp.float32).at[:, :K].set(env_W)
    ebp = jnp.full((1, H), -1e30, jnp.float32).at[0, :K].set(env_b)
    return ewp, ebp


def kernel(x, edge_index, W_in, b_in, env_W1, env_b1, conv_W1,
           env_W2, env_b2, conv_W2, W_out, b_out):
    row3 = edge_index[0].reshape(_NW, _NCH, _CH)
    col3 = edge_index[1].reshape(_NW, _NCH, _CH)
    ones16 = jnp.ones((_CH, 16), jnp.float32)
    z16 = jnp.zeros((_RPT, 16), jnp.float32)
    zH = jnp.zeros((_RPT, H), jnp.float32)

    dp = _sc_degree(col3, ones16, z16)

    ewp1, ebp1 = _pad_env(env_W1, env_b1)
    ewp2, ebp2 = _pad_env(env_W2, env_b2)
    wa1 = jnp.transpose(conv_W1[:, :H, :], (1, 0, 2)).reshape(H, K * H)
    wb1 = jnp.transpose(conv_W1[:, H:, :], (1, 0, 2)).reshape(H, K * H)
    wa2 = jnp.transpose(conv_W2[:, :H, :], (1, 0, 2)).reshape(H, K * H)
    wb2 = jnp.transpose(conv_W2[:, H:, :], (1, 0, 2)).reshape(H, K * H)
    sel = jnp.concatenate(
        [jnp.kron(jnp.eye(K, dtype=jnp.float32), jnp.ones((1, H), jnp.float32)),
         jnp.zeros((H - K, K * H), jnp.float32)], axis=0)
    wo = jnp.zeros((H, H), jnp.float32).at[:, :C].set(W_out)
    bo = jnp.zeros((1, H), jnp.float32).at[0, :C].set(b_out)

    h1, xs1 = _tc_pre(x, W_in, b_in.reshape(1, H), dp)
    agg1 = _sc_conv(xs1, row3, col3, zH)
    h2, xs2 = _tc_layer(h1, agg1, dp, ewp1, ebp1, wa1, wb1, sel)
    agg2 = _sc_conv(xs2, row3, col3, zH)
    out_pad = _tc_final(h2, agg2, dp, ewp2, ebp2, wa2, wb2, sel, wo, bo)
    return out_pad[:, :C]


# speedup vs baseline: 1.1123x; 1.0995x over previous
import functools

import jax
import jax.numpy as jnp
from jax import lax
from jax.experimental import pallas as pl
from jax.experimental.pallas import tpu as pltpu
from jax.experimental.pallas import tpu_sc as plsc

N = 10000
E = 320000
D = 128
H = 128
K = 4
C = 16

_NC = 2
_NS = 16
_NW = _NC * _NS
_EPT = E // _NW
_CH = 125
_NCH = _EPT // _CH
_RPT = N // _NS


@functools.lru_cache(maxsize=None)
def _sc_degree_kernel():
    mesh = plsc.VectorSubcoreMesh(core_axis_name="c", subcore_axis_name="s")
    return functools.partial(
        pl.kernel,
        out_type=jax.ShapeDtypeStruct((_NC, N, 16), jnp.float32),
        mesh=mesh,
        scratch_types=[
            pltpu.VMEM((_NCH, _CH), jnp.int32),
            pltpu.VMEM((_CH, 16), jnp.float32),
            pltpu.VMEM_SHARED((N, 16), jnp.float32),
            pltpu.SemaphoreType.DMA,
        ],
        compiler_params=pltpu.CompilerParams(use_tc_tiling_on_sc=False),
    )(_sc_degree_body)


def _sc_degree(col3, ones16, z16):
    return _sc_degree_kernel()(col3, ones16, z16)


def _sc_degree_body(col_hbm, ones_hbm, zrows_hbm, out_hbm, colv, onesv, acc, sem):
    cid = lax.axis_index("c")
    sid = lax.axis_index("s")
    wid = cid * _NS + sid
    pltpu.sync_copy(col_hbm.at[wid], colv)
    pltpu.sync_copy(ones_hbm, onesv)
    pltpu.sync_copy(zrows_hbm, acc.at[pl.ds(sid * _RPT, _RPT)])
    plsc.subcore_barrier()

    def body(j, carry):
        pltpu.sync_copy(onesv, acc.at[colv.at[j]], add=True)
        return carry

    lax.fori_loop(0, _NCH, body, 0)
    plsc.subcore_barrier()
    pltpu.sync_copy(acc.at[pl.ds(sid * _RPT, _RPT)],
                    out_hbm.at[cid, pl.ds(sid * _RPT, _RPT)])


@functools.lru_cache(maxsize=None)
def _sc_conv_kernel():
    mesh = plsc.VectorSubcoreMesh(core_axis_name="c", subcore_axis_name="s")
    return functools.partial(
        pl.kernel,
        out_type=jax.ShapeDtypeStruct((_NC, N, H), jnp.float32),
        mesh=mesh,
        scratch_types=[
            pltpu.VMEM((_NCH // 2, _CH), jnp.int32),
            pltpu.VMEM((_NCH // 2, _CH), jnp.int32),
            pltpu.VMEM((2, _CH, H), jnp.float32),
            pltpu.VMEM_SHARED((N, H), jnp.float32),
            pltpu.SemaphoreType.DMA,
            pltpu.SemaphoreType.DMA,
        ],
        compiler_params=pltpu.CompilerParams(use_tc_tiling_on_sc=False),
    )(_sc_conv_body)


def _sc_conv(xs, row3, col3, zH):
    return _sc_conv_kernel()(xs, row3, col3, zH)


def _sc_conv_body(xs_hbm, row_hbm, col_hbm, zrows_hbm, out_hbm,
                  rowv, colv, gbuf, acc, sem0, sem1):
    cid = lax.axis_index("c")
    sid = lax.axis_index("s")
    wid = cid * _NS + sid
    nh = _NCH // 2
    pltpu.sync_copy(zrows_hbm, acc.at[pl.ds(sid * _RPT, _RPT)])
    plsc.subcore_barrier()

    sems = (sem0, sem1)
    for half in range(2):
        pltpu.sync_copy(row_hbm.at[wid, pl.ds(half * nh, nh)], rowv)
        pltpu.sync_copy(col_hbm.at[wid, pl.ds(half * nh, nh)], colv)
        pltpu.async_copy(xs_hbm.at[rowv.at[0]], gbuf.at[0], sem0)
        pltpu.async_copy(xs_hbm.at[rowv.at[1]], gbuf.at[1], sem1)

        def body(jj, carry):
            for b in range(2):
                j = jj * 2 + b
                pltpu.make_async_copy(xs_hbm.at[rowv.at[j]], gbuf.at[b],
                                      sems[b]).wait()
                pltpu.sync_copy(gbuf.at[b], acc.at[colv.at[j]], add=True)

                @pl.when(j + 2 < nh)
                def _():
                    pltpu.async_copy(xs_hbm.at[rowv.at[j + 2]], gbuf.at[b],
                                     sems[b])
            return carry

        lax.fori_loop(0, nh // 2, body, 0)
    plsc.subcore_barrier()
    pltpu.sync_copy(acc.at[pl.ds(sid * _RPT, _RPT)],
                    out_hbm.at[cid, pl.ds(sid * _RPT, _RPT)])


_BLK = 1000
_GRID = N // _BLK


def _dis_from_parts(dp):
    deg = dp[0, :, 0:1] + dp[1, :, 0:1]
    return jnp.where(deg > 0, lax.rsqrt(deg), 0.0)


def _chunk_sum(pr):
    return pr[:, 0:128] + pr[:, 128:256] + pr[:, 256:384] + pr[:, 384:512]


def _tc_h_body(x_ref, w_ref, b_ref, h_ref):
    h_ref[...] = jnp.maximum(x_ref[...] @ w_ref[...] + b_ref[...], 0.0)


def _tc_self_body(h_ref, dp_ref, ewp_ref, ebp_ref, wb_ref, s_ref,
                  xs_ref, sp_ref, ep_ref):
    h = h_ref[...]
    dis = _dis_from_parts(dp_ref[...])
    logits = h @ ewp_ref[...] + ebp_ref[...]
    m = jnp.max(logits, axis=-1, keepdims=True)
    p = jnp.exp(logits - m)
    e = p / jnp.sum(p, axis=-1, keepdims=True)
    pb = (h @ wb_ref[...]) * (e @ s_ref[...])
    xs_ref[...] = h * dis
    sp_ref[...] = _chunk_sum(pb) + h
    ep_ref[...] = e


def _agg_mix(agg_ref, dp_ref, ep_ref, wa_ref, s_ref, sp_ref):
    agg = agg_ref[...]
    dis = _dis_from_parts(dp_ref[...])
    hi = (agg[0] + agg[1]) * dis
    pa = (hi @ wa_ref[...]) * (ep_ref[...] @ s_ref[...])
    return jnp.maximum(_chunk_sum(pa) + sp_ref[...], 0.0), dis


def _tc_post_body(agg_ref, dp_ref, ep_ref, wa_ref, s_ref, sp_ref,
                  hn_ref, xs_ref):
    hn, dis = _agg_mix(agg_ref, dp_ref, ep_ref, wa_ref, s_ref, sp_ref)
    hn_ref[...] = hn
    xs_ref[...] = hn * dis


def _tc_postf_body(agg_ref, dp_ref, ep_ref, wa_ref, s_ref, sp_ref,
                   wo_ref, bo_ref, out_ref):
    hn, _ = _agg_mix(agg_ref, dp_ref, ep_ref, wa_ref, s_ref, sp_ref)
    out_ref[...] = hn @ wo_ref[...] + bo_ref[...]


_row_spec = pl.BlockSpec((_BLK, H), lambda i: (i, 0))
_dp_spec = pl.BlockSpec((2, _BLK, 16), lambda i: (0, i, 0))
_agg_spec = pl.BlockSpec((2, _BLK, H), lambda i: (0, i, 0))
_w_spec = pl.BlockSpec((H, H), lambda i: (0, 0))
_b_spec = pl.BlockSpec((1, H), lambda i: (0, 0))
_wcat_spec = pl.BlockSpec((H, K * H), lambda i: (0, 0))
_nh_shape = jax.ShapeDtypeStruct((N, H), jnp.float32)


def _tc_h(x, w, b2):
    return pl.pallas_call(
        _tc_h_body,
        grid=(_GRID,),
        in_specs=[_row_spec, _w_spec, _b_spec],
        out_specs=_row_spec,
        out_shape=_nh_shape,
    )(x, w, b2)


def _tc_self(h, dp, ewp, ebp, wb, s):
    return pl.pallas_call(
        _tc_self_body,
        grid=(_GRID,),
        in_specs=[_row_spec, _dp_spec, _w_spec, _b_spec, _wcat_spec,
                  _wcat_spec],
        out_specs=[_row_spec] * 3,
        out_shape=[_nh_shape] * 3,
    )(h, dp, ewp, ebp, wb, s)


def _tc_post(agg, dp, ep, wa, s, sp):
    return pl.pallas_call(
        _tc_post_body,
        grid=(_GRID,),
        in_specs=[_agg_spec, _dp_spec, _row_spec, _wcat_spec, _wcat_spec,
                  _row_spec],
        out_specs=[_row_spec] * 2,
        out_shape=[_nh_shape] * 2,
    )(agg, dp, ep, wa, s, sp)


def _tc_postf(agg, dp, ep, wa, s, sp, wo, bo):
    return pl.pallas_call(
        _tc_postf_body,
        grid=(_GRID,),
        in_specs=[_agg_spec, _dp_spec, _row_spec, _wcat_spec, _wcat_spec,
                  _row_spec, _w_spec, _b_spec],
        out_specs=_row_spec,
        out_shape=_nh_shape,
    )(agg, dp, ep, wa, s, sp, wo, bo)


def _pad_env(env_W, env_b):
    ewp = jnp.zeros((H, H), jnp.float32).at[:, :K].set(env_W)
    ebp = jnp.full((1, H), -1e30, jnp.float32).at[0, :K].set(env_b)
    return ewp, ebp


def kernel(x, edge_index, W_in, b_in, env_W1, env_b1, conv_W1,
           env_W2, env_b2, conv_W2, W_out, b_out):
    row3 = edge_index[0].reshape(_NW, _NCH, _CH)
    col3 = edge_index[1].reshape(_NW, _NCH, _CH)
    ones16 = jnp.ones((_CH, 16), jnp.float32)
    z16 = jnp.zeros((_RPT, 16), jnp.float32)
    zH = jnp.zeros((_RPT, H), jnp.float32)

    dp = _sc_degree(col3, ones16, z16)

    ewp1, ebp1 = _pad_env(env_W1, env_b1)
    ewp2, ebp2 = _pad_env(env_W2, env_b2)
    wa1 = jnp.transpose(conv_W1[:, :H, :], (1, 0, 2)).reshape(H, K * H)
    wb1 = jnp.transpose(conv_W1[:, H:, :], (1, 0, 2)).reshape(H, K * H)
    wa2 = jnp.transpose(conv_W2[:, :H, :], (1, 0, 2)).reshape(H, K * H)
    wb2 = jnp.transpose(conv_W2[:, H:, :], (1, 0, 2)).reshape(H, K * H)
    sel = jnp.concatenate(
        [jnp.kron(jnp.eye(K, dtype=jnp.float32), jnp.ones((1, H), jnp.float32)),
         jnp.zeros((H - K, K * H), jnp.float32)], axis=0)
    wo = jnp.zeros((H, H), jnp.float32).at[:, :C].set(W_out)
    bo = jnp.zeros((1, H), jnp.float32).at[0, :C].set(b_out)

    h1 = _tc_h(x, W_in, b_in.reshape(1, H))
    xs1, sp1, e1p = _tc_self(h1, dp, ewp1, ebp1, wb1, sel)
    agg1 = _sc_conv(xs1, row3, col3, zH)
    h2, xs2 = _tc_post(agg1, dp, e1p, wa1, sel, sp1)
    agg2 = _sc_conv(xs2, row3, col3, zH)
    _, sp2, e2p = _tc_self(h2, dp, ewp2, ebp2, wb2, sel)
    out_pad = _tc_postf(agg2, dp, e2p, wa2, sel, sp2, wo, bo)
    return out_pad[:, :C]


# speedup vs baseline: 1.1791x; 1.0601x over previous
import functools

import jax
import jax.numpy as jnp
from jax import lax
from jax.experimental import pallas as pl
from jax.experimental.pallas import tpu as pltpu
from jax.experimental.pallas import tpu_sc as plsc

N = 10000
E = 320000
D = 128
H = 128
K = 4
C = 16

_NC = 2
_NS = 16
_NW = _NC * _NS
_EPT = E // _NW
_CH = 125
_NCH = _EPT // _CH
_RPT = N // _NS


@functools.lru_cache(maxsize=None)
def _sc_degree_kernel():
    mesh = plsc.VectorSubcoreMesh(core_axis_name="c", subcore_axis_name="s")
    return functools.partial(
        pl.kernel,
        out_type=jax.ShapeDtypeStruct((_NC, N, 16), jnp.float32),
        mesh=mesh,
        scratch_types=[
            pltpu.VMEM((_NCH, _CH), jnp.int32),
            pltpu.VMEM((_CH, 16), jnp.float32),
            pltpu.VMEM_SHARED((N, 16), jnp.float32),
            pltpu.SemaphoreType.DMA,
        ],
        compiler_params=pltpu.CompilerParams(use_tc_tiling_on_sc=False),
    )(_sc_degree_body)


def _sc_degree(col3, ones16, z16):
    return _sc_degree_kernel()(col3, ones16, z16)


def _sc_degree_body(col_hbm, ones_hbm, zrows_hbm, out_hbm, colv, onesv, acc, sem):
    cid = lax.axis_index("c")
    sid = lax.axis_index("s")
    wid = cid * _NS + sid
    pltpu.sync_copy(col_hbm.at[wid], colv)
    pltpu.sync_copy(ones_hbm, onesv)
    pltpu.sync_copy(zrows_hbm, acc.at[pl.ds(sid * _RPT, _RPT)])
    plsc.subcore_barrier()

    def body(j, carry):
        pltpu.sync_copy(onesv, acc.at[colv.at[j]], add=True)
        return carry

    lax.fori_loop(0, _NCH, body, 0)
    plsc.subcore_barrier()
    pltpu.sync_copy(acc.at[pl.ds(sid * _RPT, _RPT)],
                    out_hbm.at[cid, pl.ds(sid * _RPT, _RPT)])


@functools.lru_cache(maxsize=None)
def _sc_conv_kernel():
    mesh = plsc.VectorSubcoreMesh(core_axis_name="c", subcore_axis_name="s")
    return functools.partial(
        pl.kernel,
        out_type=jax.ShapeDtypeStruct((_NC, N, H), jnp.bfloat16),
        mesh=mesh,
        scratch_types=[
            pltpu.VMEM((_NCH // 2, _CH), jnp.int32),
            pltpu.VMEM((_NCH // 2, _CH), jnp.int32),
            pltpu.VMEM((2, _CH, H), jnp.bfloat16),
            pltpu.VMEM_SHARED((N, H), jnp.bfloat16),
            pltpu.SemaphoreType.DMA,
            pltpu.SemaphoreType.DMA,
        ],
        compiler_params=pltpu.CompilerParams(use_tc_tiling_on_sc=False),
    )(_sc_conv_body)


def _sc_conv(xs, row3, col3, zH):
    return _sc_conv_kernel()(xs, row3, col3, zH)


def _sc_conv_body(xs_hbm, row_hbm, col_hbm, zrows_hbm, out_hbm,
                  rowv, colv, gbuf, acc, sem0, sem1):
    cid = lax.axis_index("c")
    sid = lax.axis_index("s")
    wid = cid * _NS + sid
    nh = _NCH // 2
    pltpu.sync_copy(zrows_hbm, acc.at[pl.ds(sid * _RPT, _RPT)])
    plsc.subcore_barrier()

    sems = (sem0, sem1)
    for half in range(2):
        pltpu.sync_copy(row_hbm.at[wid, pl.ds(half * nh, nh)], rowv)
        pltpu.sync_copy(col_hbm.at[wid, pl.ds(half * nh, nh)], colv)
        pltpu.async_copy(xs_hbm.at[rowv.at[0]], gbuf.at[0], sem0)
        pltpu.async_copy(xs_hbm.at[rowv.at[1]], gbuf.at[1], sem1)

        def body(jj, carry):
            for b in range(2):
                j = jj * 2 + b
                pltpu.make_async_copy(xs_hbm.at[rowv.at[j]], gbuf.at[b],
                                      sems[b]).wait()
                pltpu.sync_copy(gbuf.at[b], acc.at[colv.at[j]], add=True)

                @pl.when(j + 2 < nh)
                def _():
                    pltpu.async_copy(xs_hbm.at[rowv.at[j + 2]], gbuf.at[b],
                                     sems[b])
            return carry

        lax.fori_loop(0, nh // 2, body, 0)
    plsc.subcore_barrier()
    pltpu.sync_copy(acc.at[pl.ds(sid * _RPT, _RPT)],
                    out_hbm.at[cid, pl.ds(sid * _RPT, _RPT)])


_BLK = 1000
_GRID = N // _BLK


def _dis_from_parts(dp):
    deg = dp[0, :, 0:1] + dp[1, :, 0:1]
    return jnp.where(deg > 0, lax.rsqrt(deg), 0.0)


def _chunk_sum(pr):
    return pr[:, 0:128] + pr[:, 128:256] + pr[:, 256:384] + pr[:, 384:512]


def _tc_h_body(x_ref, w_ref, b_ref, h_ref):
    h_ref[...] = jnp.maximum(x_ref[...] @ w_ref[...] + b_ref[...], 0.0)


def _tc_self_body(h_ref, dp_ref, ewp_ref, ebp_ref, wb_ref, s_ref,
                  xs_ref, sp_ref, ep_ref):
    h = h_ref[...]
    dis = _dis_from_parts(dp_ref[...])
    logits = h @ ewp_ref[...] + ebp_ref[...]
    m = jnp.max(logits, axis=-1, keepdims=True)
    p = jnp.exp(logits - m)
    e = p / jnp.sum(p, axis=-1, keepdims=True)
    pb = (h @ wb_ref[...]) * (e @ s_ref[...])
    xs_ref[...] = (h * dis).astype(jnp.bfloat16)
    sp_ref[...] = _chunk_sum(pb) + h
    ep_ref[...] = e


def _agg_mix(agg_ref, dp_ref, ep_ref, wa_ref, s_ref, sp_ref):
    agg = agg_ref[...].astype(jnp.float32)
    dis = _dis_from_parts(dp_ref[...])
    hi = (agg[0] + agg[1]) * dis
    pa = (hi @ wa_ref[...]) * (ep_ref[...] @ s_ref[...])
    return jnp.maximum(_chunk_sum(pa) + sp_ref[...], 0.0), dis


def _tc_post_body(agg_ref, dp_ref, ep_ref, wa_ref, s_ref, sp_ref,
                  hn_ref, xs_ref):
    hn, dis = _agg_mix(agg_ref, dp_ref, ep_ref, wa_ref, s_ref, sp_ref)
    hn_ref[...] = hn
    xs_ref[...] = (hn * dis).astype(jnp.bfloat16)


def _tc_postf_body(agg_ref, dp_ref, ep_ref, wa_ref, s_ref, sp_ref,
                   wo_ref, bo_ref, out_ref):
    hn, _ = _agg_mix(agg_ref, dp_ref, ep_ref, wa_ref, s_ref, sp_ref)
    out_ref[...] = hn @ wo_ref[...] + bo_ref[...]


_row_spec = pl.BlockSpec((_BLK, H), lambda i: (i, 0))
_dp_spec = pl.BlockSpec((2, _BLK, 16), lambda i: (0, i, 0))
_agg_spec = pl.BlockSpec((2, _BLK, H), lambda i: (0, i, 0))
_w_spec = pl.BlockSpec((H, H), lambda i: (0, 0))
_b_spec = pl.BlockSpec((1, H), lambda i: (0, 0))
_wcat_spec = pl.BlockSpec((H, K * H), lambda i: (0, 0))
_nh_shape = jax.ShapeDtypeStruct((N, H), jnp.float32)
_xs_bshape = jax.ShapeDtypeStruct((N, H), jnp.bfloat16)


def _tc_h(x, w, b2):
    return pl.pallas_call(
        _tc_h_body,
        grid=(_GRID,),
        in_specs=[_row_spec, _w_spec, _b_spec],
        out_specs=_row_spec,
        out_shape=_nh_shape,
    )(x, w, b2)


def _tc_self(h, dp, ewp, ebp, wb, s):
    return pl.pallas_call(
        _tc_self_body,
        grid=(_GRID,),
        in_specs=[_row_spec, _dp_spec, _w_spec, _b_spec, _wcat_spec,
                  _wcat_spec],
        out_specs=[_row_spec] * 3,
        out_shape=[_xs_bshape, _nh_shape, _nh_shape],
    )(h, dp, ewp, ebp, wb, s)


def _tc_post(agg, dp, ep, wa, s, sp):
    return pl.pallas_call(
        _tc_post_body,
        grid=(_GRID,),
        in_specs=[_agg_spec, _dp_spec, _row_spec, _wcat_spec, _wcat_spec,
                  _row_spec],
        out_specs=[_row_spec] * 2,
        out_shape=[_nh_shape, _xs_bshape],
    )(agg, dp, ep, wa, s, sp)


def _tc_postf(agg, dp, ep, wa, s, sp, wo, bo):
    return pl.pallas_call(
        _tc_postf_body,
        grid=(_GRID,),
        in_specs=[_agg_spec, _dp_spec, _row_spec, _wcat_spec, _wcat_spec,
                  _row_spec, _w_spec, _b_spec],
        out_specs=_row_spec,
        out_shape=_nh_shape,
    )(agg, dp, ep, wa, s, sp, wo, bo)


def _pad_env(env_W, env_b):
    ewp = jnp.zeros((H, H), jnp.float32).at[:, :K].set(env_W)
    ebp = jnp.full((1, H), -1e30, jnp.float32).at[0, :K].set(env_b)
    return ewp, ebp


def kernel(x, edge_index, W_in, b_in, env_W1, env_b1, conv_W1,
           env_W2, env_b2, conv_W2, W_out, b_out):
    row3 = edge_index[0].reshape(_NW, _NCH, _CH)
    col3 = edge_index[1].reshape(_NW, _NCH, _CH)
    ones16 = jnp.ones((_CH, 16), jnp.float32)
    z16 = jnp.zeros((_RPT, 16), jnp.float32)
    zH = jnp.zeros((_RPT, H), jnp.bfloat16)

    dp = _sc_degree(col3, ones16, z16)

    ewp1, ebp1 = _pad_env(env_W1, env_b1)
    ewp2, ebp2 = _pad_env(env_W2, env_b2)
    wa1 = jnp.transpose(conv_W1[:, :H, :], (1, 0, 2)).reshape(H, K * H)
    wb1 = jnp.transpose(conv_W1[:, H:, :], (1, 0, 2)).reshape(H, K * H)
    wa2 = jnp.transpose(conv_W2[:, :H, :], (1, 0, 2)).reshape(H, K * H)
    wb2 = jnp.transpose(conv_W2[:, H:, :], (1, 0, 2)).reshape(H, K * H)
    sel = jnp.concatenate(
        [jnp.kron(jnp.eye(K, dtype=jnp.float32), jnp.ones((1, H), jnp.float32)),
         jnp.zeros((H - K, K * H), jnp.float32)], axis=0)
    wo = jnp.zeros((H, H), jnp.float32).at[:, :C].set(W_out)
    bo = jnp.zeros((1, H), jnp.float32).at[0, :C].set(b_out)

    h1 = _tc_h(x, W_in, b_in.reshape(1, H))
    xs1, sp1, e1p = _tc_self(h1, dp, ewp1, ebp1, wb1, sel)
    agg1 = _sc_conv(xs1, row3, col3, zH)
    h2, xs2 = _tc_post(agg1, dp, e1p, wa1, sel, sp1)
    agg2 = _sc_conv(xs2, row3, col3, zH)
    _, sp2, e2p = _tc_self(h2, dp, ewp2, ebp2, wb2, sel)
    out_pad = _tc_postf(agg2, dp, e2p, wa2, sel, sp2, wo, bo)
    return out_pad[:, :C]


# speedup vs baseline: 1.1853x; 1.0053x over previous
import functools

import jax
import jax.numpy as jnp
from jax import lax
from jax.experimental import pallas as pl
from jax.experimental.pallas import tpu as pltpu
from jax.experimental.pallas import tpu_sc as plsc

N = 10000
E = 320000
D = 128
H = 128
K = 4
C = 16

_NC = 2
_NS = 16
_NW = _NC * _NS
_EPT = E // _NW
_CH = 125
_NCH = _EPT // _CH
_RPT = N // _NS


@functools.lru_cache(maxsize=None)
def _sc_degree_kernel():
    mesh = plsc.VectorSubcoreMesh(core_axis_name="c", subcore_axis_name="s")
    return functools.partial(
        pl.kernel,
        out_type=jax.ShapeDtypeStruct((_NC, N, 16), jnp.float32),
        mesh=mesh,
        scratch_types=[
            pltpu.VMEM((_NCH, _CH), jnp.int32),
            pltpu.VMEM((_CH, 16), jnp.float32),
            pltpu.VMEM_SHARED((N, 16), jnp.float32),
            pltpu.SemaphoreType.DMA,
        ],
        compiler_params=pltpu.CompilerParams(use_tc_tiling_on_sc=False),
    )(_sc_degree_body)


def _sc_degree(col3, ones16, z16):
    return _sc_degree_kernel()(col3, ones16, z16)


def _sc_degree_body(col_hbm, ones_hbm, zrows_hbm, out_hbm, colv, onesv, acc, sem):
    cid = lax.axis_index("c")
    sid = lax.axis_index("s")
    wid = cid * _NS + sid
    pltpu.sync_copy(col_hbm.at[wid], colv)
    pltpu.sync_copy(ones_hbm, onesv)
    pltpu.sync_copy(zrows_hbm, acc.at[pl.ds(sid * _RPT, _RPT)])
    plsc.subcore_barrier()

    def body(j, carry):
        pltpu.sync_copy(onesv, acc.at[colv.at[j]], add=True)
        return carry

    lax.fori_loop(0, _NCH, body, 0)
    plsc.subcore_barrier()
    pltpu.sync_copy(acc.at[pl.ds(sid * _RPT, _RPT)],
                    out_hbm.at[cid, pl.ds(sid * _RPT, _RPT)])


@functools.lru_cache(maxsize=None)
def _sc_conv_kernel():
    mesh = plsc.VectorSubcoreMesh(core_axis_name="c", subcore_axis_name="s")
    return functools.partial(
        pl.kernel,
        out_type=jax.ShapeDtypeStruct((_NC, N, H), jnp.bfloat16),
        mesh=mesh,
        scratch_types=[
            pltpu.VMEM((_NCH // 2, _CH), jnp.int32),
            pltpu.VMEM((_NCH // 2, _CH), jnp.int32),
            pltpu.VMEM((2, _CH, H), jnp.bfloat16),
            pltpu.VMEM_SHARED((N, H), jnp.bfloat16),
            pltpu.SemaphoreType.DMA,
            pltpu.SemaphoreType.DMA,
        ],
        compiler_params=pltpu.CompilerParams(use_tc_tiling_on_sc=False),
    )(_sc_conv_body)


def _sc_conv(xs, row3, col3, zH):
    return _sc_conv_kernel()(xs, row3, col3, zH)


def _sc_conv_body(xs_hbm, row_hbm, col_hbm, zrows_hbm, out_hbm,
                  rowv, colv, gbuf, acc, sem0, sem1):
    cid = lax.axis_index("c")
    sid = lax.axis_index("s")
    wid = cid * _NS + sid
    nh = _NCH // 2
    sems = (sem0, sem1)
    pltpu.sync_copy(row_hbm.at[wid, pl.ds(0, nh)], rowv)
    pltpu.sync_copy(col_hbm.at[wid, pl.ds(0, nh)], colv)
    pltpu.async_copy(xs_hbm.at[rowv.at[0]], gbuf.at[0], sem0)
    pltpu.async_copy(xs_hbm.at[rowv.at[1]], gbuf.at[1], sem1)
    pltpu.sync_copy(zrows_hbm, acc.at[pl.ds(sid * _RPT, _RPT)])
    plsc.subcore_barrier()

    for half in range(2):
        if half:
            pltpu.sync_copy(row_hbm.at[wid, pl.ds(half * nh, nh)], rowv)
            pltpu.sync_copy(col_hbm.at[wid, pl.ds(half * nh, nh)], colv)
            pltpu.async_copy(xs_hbm.at[rowv.at[0]], gbuf.at[0], sem0)
            pltpu.async_copy(xs_hbm.at[rowv.at[1]], gbuf.at[1], sem1)

        def body(jj, carry):
            for b in range(2):
                j = jj * 2 + b
                pltpu.make_async_copy(xs_hbm.at[rowv.at[j]], gbuf.at[b],
                                      sems[b]).wait()
                pltpu.sync_copy(gbuf.at[b], acc.at[colv.at[j]], add=True)

                @pl.when(j + 2 < nh)
                def _():
                    pltpu.async_copy(xs_hbm.at[rowv.at[j + 2]], gbuf.at[b],
                                     sems[b])
            return carry

        lax.fori_loop(0, nh // 2, body, 0)
    plsc.subcore_barrier()
    pltpu.sync_copy(acc.at[pl.ds(sid * _RPT, _RPT)],
                    out_hbm.at[cid, pl.ds(sid * _RPT, _RPT)])


_BLK = 1000
_GRID = N // _BLK


def _dis_from_parts(dp):
    deg = dp[0, :, 0:1] + dp[1, :, 0:1]
    return jnp.where(deg > 0, lax.rsqrt(deg), 0.0)


def _chunk_sum(pr):
    return pr[:, 0:128] + pr[:, 128:256] + pr[:, 256:384] + pr[:, 384:512]


def _tc_h_body(x_ref, w_ref, b_ref, h_ref):
    h_ref[...] = jnp.maximum(x_ref[...] @ w_ref[...] + b_ref[...], 0.0)


def _softmax_pad(h, ewp, ebp):
    logits = h @ ewp + ebp
    m = jnp.max(logits, axis=-1, keepdims=True)
    p = jnp.exp(logits - m)
    return p / jnp.sum(p, axis=-1, keepdims=True)


def _tc_self_body(h_ref, dp_ref, ewp_ref, ebp_ref, wb_ref, s_ref,
                  xs_ref, sp_ref):
    h = h_ref[...]
    dis = _dis_from_parts(dp_ref[...])
    e = _softmax_pad(h, ewp_ref[...], ebp_ref[...])
    pb = (h @ wb_ref[...]) * (e @ s_ref[...])
    xs_ref[...] = (h * dis).astype(jnp.bfloat16)
    sp_ref[...] = _chunk_sum(pb) + h


def _agg_mix(agg_ref, dp_ref, h_ref, ewp_ref, ebp_ref, wa_ref, s_ref,
             sp_ref):
    agg = agg_ref[...].astype(jnp.float32)
    dis = _dis_from_parts(dp_ref[...])
    hi = (agg[0] + agg[1]) * dis
    e = _softmax_pad(h_ref[...], ewp_ref[...], ebp_ref[...])
    pa = (hi @ wa_ref[...]) * (e @ s_ref[...])
    return jnp.maximum(_chunk_sum(pa) + sp_ref[...], 0.0), dis


def _tc_post_body(agg_ref, dp_ref, h_ref, ewp_ref, ebp_ref, wa_ref, s_ref,
                  sp_ref, hn_ref, xs_ref):
    hn, dis = _agg_mix(agg_ref, dp_ref, h_ref, ewp_ref, ebp_ref, wa_ref,
                       s_ref, sp_ref)
    hn_ref[...] = hn
    xs_ref[...] = (hn * dis).astype(jnp.bfloat16)


def _tc_postf_body(agg_ref, dp_ref, h_ref, ewp_ref, ebp_ref, wa_ref, s_ref,
                   sp_ref, wo_ref, bo_ref, out_ref):
    hn, _ = _agg_mix(agg_ref, dp_ref, h_ref, ewp_ref, ebp_ref, wa_ref,
                     s_ref, sp_ref)
    out_ref[...] = hn @ wo_ref[...] + bo_ref[...]


_row_spec = pl.BlockSpec((_BLK, H), lambda i: (i, 0))
_dp_spec = pl.BlockSpec((2, _BLK, 16), lambda i: (0, i, 0))
_agg_spec = pl.BlockSpec((2, _BLK, H), lambda i: (0, i, 0))
_w_spec = pl.BlockSpec((H, H), lambda i: (0, 0))
_b_spec = pl.BlockSpec((1, H), lambda i: (0, 0))
_wcat_spec = pl.BlockSpec((H, K * H), lambda i: (0, 0))
_nh_shape = jax.ShapeDtypeStruct((N, H), jnp.float32)
_xs_bshape = jax.ShapeDtypeStruct((N, H), jnp.bfloat16)


def _tc_h(x, w, b2):
    return pl.pallas_call(
        _tc_h_body,
        grid=(_GRID,),
        in_specs=[_row_spec, _w_spec, _b_spec],
        out_specs=_row_spec,
        out_shape=_nh_shape,
    )(x, w, b2)


def _tc_self(h, dp, ewp, ebp, wb, s):
    return pl.pallas_call(
        _tc_self_body,
        grid=(_GRID,),
        in_specs=[_row_spec, _dp_spec, _w_spec, _b_spec, _wcat_spec,
                  _wcat_spec],
        out_specs=[_row_spec] * 2,
        out_shape=[_xs_bshape, _nh_shape],
    )(h, dp, ewp, ebp, wb, s)


def _tc_post(agg, dp, h, ewp, ebp, wa, s, sp):
    return pl.pallas_call(
        _tc_post_body,
        grid=(_GRID,),
        in_specs=[_agg_spec, _dp_spec, _row_spec, _w_spec, _b_spec,
                  _wcat_spec, _wcat_spec, _row_spec],
        out_specs=[_row_spec] * 2,
        out_shape=[_nh_shape, _xs_bshape],
    )(agg, dp, h, ewp, ebp, wa, s, sp)


def _tc_postf(agg, dp, h, ewp, ebp, wa, s, sp, wo, bo):
    return pl.pallas_call(
        _tc_postf_body,
        grid=(_GRID,),
        in_specs=[_agg_spec, _dp_spec, _row_spec, _w_spec, _b_spec,
                  _wcat_spec, _wcat_spec, _row_spec, _w_spec, _b_spec],
        out_specs=_row_spec,
        out_shape=_nh_shape,
    )(agg, dp, h, ewp, ebp, wa, s, sp, wo, bo)


def _pad_env(env_W, env_b):
    ewp = jnp.zeros((H, H), jnp.float32).at[:, :K].set(env_W)
    ebp = jnp.full((1, H), -1e30, jnp.float32).at[0, :K].set(env_b)
    return ewp, ebp


def kernel(x, edge_index, W_in, b_in, env_W1, env_b1, conv_W1,
           env_W2, env_b2, conv_W2, W_out, b_out):
    row3 = edge_index[0].reshape(_NW, _NCH, _CH)
    col3 = edge_index[1].reshape(_NW, _NCH, _CH)
    ones16 = jnp.ones((_CH, 16), jnp.float32)
    z16 = jnp.zeros((_RPT, 16), jnp.float32)
    zH = jnp.zeros((_RPT, H), jnp.bfloat16)

    dp = _sc_degree(col3, ones16, z16)

    ewp1, ebp1 = _pad_env(env_W1, env_b1)
    ewp2, ebp2 = _pad_env(env_W2, env_b2)
    wa1 = jnp.transpose(conv_W1[:, :H, :], (1, 0, 2)).reshape(H, K * H)
    wb1 = jnp.transpose(conv_W1[:, H:, :], (1, 0, 2)).reshape(H, K * H)
    wa2 = jnp.transpose(conv_W2[:, :H, :], (1, 0, 2)).reshape(H, K * H)
    wb2 = jnp.transpose(conv_W2[:, H:, :], (1, 0, 2)).reshape(H, K * H)
    sel = jnp.concatenate(
        [jnp.kron(jnp.eye(K, dtype=jnp.float32), jnp.ones((1, H), jnp.float32)),
         jnp.zeros((H - K, K * H), jnp.float32)], axis=0)
    wo = jnp.zeros((H, H), jnp.float32).at[:, :C].set(W_out)
    bo = jnp.zeros((1, H), jnp.float32).at[0, :C].set(b_out)

    h1 = _tc_h(x, W_in, b_in.reshape(1, H))
    xs1, sp1 = _tc_self(h1, dp, ewp1, ebp1, wb1, sel)
    agg1 = _sc_conv(xs1, row3, col3, zH)
    h2, xs2 = _tc_post(agg1, dp, h1, ewp1, ebp1, wa1, sel, sp1)
    agg2 = _sc_conv(xs2, row3, col3, zH)
    _, sp2 = _tc_self(h2, dp, ewp2, ebp2, wb2, sel)
    out_pad = _tc_postf(agg2, dp, h2, ewp2, ebp2, wa2, sel, sp2, wo, bo)
    return out_pad[:, :C]


# speedup vs baseline: 1.2431x; 1.0488x over previous
import functools

import jax
import jax.numpy as jnp
from jax import lax
from jax.experimental import pallas as pl
from jax.experimental.pallas import tpu as pltpu
from jax.experimental.pallas import tpu_sc as plsc

N = 10000
E = 320000
D = 128
H = 128
K = 4
C = 16

_NC = 2
_NS = 16
_NW = _NC * _NS
_EPT = E // _NW
_CH = 125
_NCH = _EPT // _CH
_RPT = N // _NS


@functools.lru_cache(maxsize=None)
def _sc_degree_kernel():
    mesh = plsc.VectorSubcoreMesh(core_axis_name="c", subcore_axis_name="s")
    return functools.partial(
        pl.kernel,
        out_type=jax.ShapeDtypeStruct((_NC, N, 16), jnp.float32),
        mesh=mesh,
        scratch_types=[
            pltpu.VMEM((_NCH, _CH), jnp.int32),
            pltpu.VMEM((_CH, 16), jnp.float32),
            pltpu.VMEM_SHARED((N, 16), jnp.float32),
            pltpu.SemaphoreType.DMA,
        ],
        compiler_params=pltpu.CompilerParams(use_tc_tiling_on_sc=False),
    )(_sc_degree_body)


def _sc_degree(eidx4, ones16, z16):
    return _sc_degree_kernel()(eidx4, ones16, z16)


def _sc_degree_body(eidx_hbm, ones_hbm, zrows_hbm, out_hbm, colv, onesv, acc, sem):
    cid = lax.axis_index("c")
    sid = lax.axis_index("s")
    wid = cid * _NS + sid
    pltpu.sync_copy(eidx_hbm.at[1, wid], colv)
    pltpu.sync_copy(ones_hbm, onesv)
    pltpu.sync_copy(zrows_hbm, acc.at[pl.ds(sid * _RPT, _RPT)])
    plsc.subcore_barrier()

    def body(j, carry):
        pltpu.sync_copy(onesv, acc.at[colv.at[j]], add=True)
        return carry

    lax.fori_loop(0, _NCH, body, 0)
    plsc.subcore_barrier()
    pltpu.sync_copy(acc.at[pl.ds(sid * _RPT, _RPT)],
                    out_hbm.at[cid, pl.ds(sid * _RPT, _RPT)])


@functools.lru_cache(maxsize=None)
def _sc_conv_kernel():
    mesh = plsc.VectorSubcoreMesh(core_axis_name="c", subcore_axis_name="s")
    return functools.partial(
        pl.kernel,
        out_type=jax.ShapeDtypeStruct((_NC, N, H), jnp.bfloat16),
        mesh=mesh,
        scratch_types=[
            pltpu.VMEM((_NCH // 2, _CH), jnp.int32),
            pltpu.VMEM((_NCH // 2, _CH), jnp.int32),
            pltpu.VMEM((2, _CH, H), jnp.bfloat16),
            pltpu.VMEM_SHARED((N, H), jnp.bfloat16),
            pltpu.SemaphoreType.DMA,
            pltpu.SemaphoreType.DMA,
        ],
        compiler_params=pltpu.CompilerParams(use_tc_tiling_on_sc=False),
    )(_sc_conv_body)


def _sc_conv(xs, eidx4, zH):
    return _sc_conv_kernel()(xs, eidx4, zH)


def _sc_conv_body(xs_hbm, eidx_hbm, zrows_hbm, out_hbm,
                  rowv, colv, gbuf, acc, sem0, sem1):
    cid = lax.axis_index("c")
    sid = lax.axis_index("s")
    wid = cid * _NS + sid
    nh = _NCH // 2
    sems = (sem0, sem1)
    pltpu.sync_copy(eidx_hbm.at[0, wid, pl.ds(0, nh)], rowv)
    pltpu.sync_copy(eidx_hbm.at[1, wid, pl.ds(0, nh)], colv)
    pltpu.async_copy(xs_hbm.at[rowv.at[0]], gbuf.at[0], sem0)
    pltpu.async_copy(xs_hbm.at[rowv.at[1]], gbuf.at[1], sem1)
    pltpu.sync_copy(zrows_hbm, acc.at[pl.ds(sid * _RPT, _RPT)])
    plsc.subcore_barrier()

    for half in range(2):
        if half:
            pltpu.sync_copy(eidx_hbm.at[0, wid, pl.ds(half * nh, nh)], rowv)
            pltpu.sync_copy(eidx_hbm.at[1, wid, pl.ds(half * nh, nh)], colv)
            pltpu.async_copy(xs_hbm.at[rowv.at[0]], gbuf.at[0], sem0)
            pltpu.async_copy(xs_hbm.at[rowv.at[1]], gbuf.at[1], sem1)

        def body(jj, carry):
            for b in range(2):
                j = jj * 2 + b
                pltpu.make_async_copy(xs_hbm.at[rowv.at[j]], gbuf.at[b],
                                      sems[b]).wait()
                pltpu.sync_copy(gbuf.at[b], acc.at[colv.at[j]], add=True)

                @pl.when(j + 2 < nh)
                def _():
                    pltpu.async_copy(xs_hbm.at[rowv.at[j + 2]], gbuf.at[b],
                                     sems[b])
            return carry

        lax.fori_loop(0, nh // 2, body, 0)
    plsc.subcore_barrier()
    pltpu.sync_copy(acc.at[pl.ds(sid * _RPT, _RPT)],
                    out_hbm.at[cid, pl.ds(sid * _RPT, _RPT)])


_BLK = 2000
_GRID = N // _BLK


def _dis_from_parts(dp):
    deg = dp[0, :, 0:1] + dp[1, :, 0:1]
    return jnp.where(deg > 0, lax.rsqrt(deg), 0.0)


def _chunk_sum(pr):
    return pr[:, 0:128] + pr[:, 128:256] + pr[:, 256:384] + pr[:, 384:512]


def _tc_h_body(x_ref, w_ref, b_ref, h_ref):
    h_ref[...] = jnp.maximum(x_ref[...] @ w_ref[...] + b_ref[...], 0.0)


def _softmax_pad(h, ewp, ebp):
    logits = h @ ewp + ebp
    m = jnp.max(logits, axis=-1, keepdims=True)
    p = jnp.exp(logits - m)
    return p / jnp.sum(p, axis=-1, keepdims=True)


def _tc_self_body(h_ref, dp_ref, ewp_ref, ebp_ref, wb_ref, s_ref,
                  xs_ref, sp_ref):
    h = h_ref[...]
    dis = _dis_from_parts(dp_ref[...])
    e = _softmax_pad(h, ewp_ref[...], ebp_ref[...])
    pb = (h @ wb_ref[...]) * (e @ s_ref[...])
    xs_ref[...] = (h * dis).astype(jnp.bfloat16)
    sp_ref[...] = _chunk_sum(pb) + h


def _agg_mix(agg_ref, dp_ref, h_ref, ewp_ref, ebp_ref, wa_ref, s_ref,
             sp_ref):
    agg = agg_ref[...].astype(jnp.float32)
    dis = _dis_from_parts(dp_ref[...])
    hi = (agg[0] + agg[1]) * dis
    e = _softmax_pad(h_ref[...], ewp_ref[...], ebp_ref[...])
    pa = (hi @ wa_ref[...]) * (e @ s_ref[...])
    return jnp.maximum(_chunk_sum(pa) + sp_ref[...], 0.0), dis


def _tc_post_body(agg_ref, dp_ref, h_ref, ewp_ref, ebp_ref, wa_ref, s_ref,
                  sp_ref, hn_ref, xs_ref):
    hn, dis = _agg_mix(agg_ref, dp_ref, h_ref, ewp_ref, ebp_ref, wa_ref,
                       s_ref, sp_ref)
    hn_ref[...] = hn
    xs_ref[...] = (hn * dis).astype(jnp.bfloat16)


def _tc_postf_body(agg_ref, dp_ref, h_ref, ewp_ref, ebp_ref, wa_ref, s_ref,
                   sp_ref, wo_ref, bo_ref, out_ref):
    hn, _ = _agg_mix(agg_ref, dp_ref, h_ref, ewp_ref, ebp_ref, wa_ref,
                     s_ref, sp_ref)
    out_ref[...] = hn @ wo_ref[...] + bo_ref[...]


_row_spec = pl.BlockSpec((_BLK, H), lambda i: (i, 0))
_dp_spec = pl.BlockSpec((2, _BLK, 16), lambda i: (0, i, 0))
_agg_spec = pl.BlockSpec((2, _BLK, H), lambda i: (0, i, 0))
_w_spec = pl.BlockSpec((H, H), lambda i: (0, 0))
_b_spec = pl.BlockSpec((1, H), lambda i: (0, 0))
_wcat_spec = pl.BlockSpec((H, K * H), lambda i: (0, 0))
_nh_shape = jax.ShapeDtypeStruct((N, H), jnp.float32)
_xs_bshape = jax.ShapeDtypeStruct((N, H), jnp.bfloat16)


def _tc_h(x, w, b2):
    return pl.pallas_call(
        _tc_h_body,
        grid=(_GRID,),
        in_specs=[_row_spec, _w_spec, _b_spec],
        out_specs=_row_spec,
        out_shape=_nh_shape,
    )(x, w, b2)


def _tc_self(h, dp, ewp, ebp, wb, s):
    return pl.pallas_call(
        _tc_self_body,
        grid=(_GRID,),
        in_specs=[_row_spec, _dp_spec, _w_spec, _b_spec, _wcat_spec,
                  _wcat_spec],
        out_specs=[_row_spec] * 2,
        out_shape=[_xs_bshape, _nh_shape],
    )(h, dp, ewp, ebp, wb, s)


def _tc_post(agg, dp, h, ewp, ebp, wa, s, sp):
    return pl.pallas_call(
        _tc_post_body,
        grid=(_GRID,),
        in_specs=[_agg_spec, _dp_spec, _row_spec, _w_spec, _b_spec,
                  _wcat_spec, _wcat_spec, _row_spec],
        out_specs=[_row_spec] * 2,
        out_shape=[_nh_shape, _xs_bshape],
    )(agg, dp, h, ewp, ebp, wa, s, sp)


def _tc_postf(agg, dp, h, ewp, ebp, wa, s, sp, wo, bo):
    return pl.pallas_call(
        _tc_postf_body,
        grid=(_GRID,),
        in_specs=[_agg_spec, _dp_spec, _row_spec, _w_spec, _b_spec,
                  _wcat_spec, _wcat_spec, _row_spec, _w_spec, _b_spec],
        out_specs=_row_spec,
        out_shape=_nh_shape,
    )(agg, dp, h, ewp, ebp, wa, s, sp, wo, bo)


def _pad_env(env_W, env_b):
    ewp = jnp.zeros((H, H), jnp.float32).at[:, :K].set(env_W)
    ebp = jnp.full((1, H), -1e30, jnp.float32).at[0, :K].set(env_b)
    return ewp, ebp


def kernel(x, edge_index, W_in, b_in, env_W1, env_b1, conv_W1,
           env_W2, env_b2, conv_W2, W_out, b_out):
    eidx4 = edge_index.reshape(2, _NW, _NCH, _CH)
    ones16 = jnp.ones((_CH, 16), jnp.float32)
    z16 = jnp.zeros((_RPT, 16), jnp.float32)
    zH = jnp.zeros((_RPT, H), jnp.bfloat16)

    dp = _sc_degree(eidx4, ones16, z16)

    ewp1, ebp1 = _pad_env(env_W1, env_b1)
    ewp2, ebp2 = _pad_env(env_W2, env_b2)
    wa1 = jnp.transpose(conv_W1[:, :H, :], (1, 0, 2)).reshape(H, K * H)
    wb1 = jnp.transpose(conv_W1[:, H:, :], (1, 0, 2)).reshape(H, K * H)
    wa2 = jnp.transpose(conv_W2[:, :H, :], (1, 0, 2)).reshape(H, K * H)
    wb2 = jnp.transpose(conv_W2[:, H:, :], (1, 0, 2)).reshape(H, K * H)
    sel = jnp.concatenate(
        [jnp.kron(jnp.eye(K, dtype=jnp.float32), jnp.ones((1, H), jnp.float32)),
         jnp.zeros((H - K, K * H), jnp.float32)], axis=0)
    wo = jnp.zeros((H, H), jnp.float32).at[:, :C].set(W_out)
    bo = jnp.zeros((1, H), jnp.float32).at[0, :C].set(b_out)

    h1 = _tc_h(x, W_in, b_in.reshape(1, H))
    xs1, sp1 = _tc_self(h1, dp, ewp1, ebp1, wb1, sel)
    agg1 = _sc_conv(xs1, eidx4, zH)
    h2, xs2 = _tc_post(agg1, dp, h1, ewp1, ebp1, wa1, sel, sp1)
    agg2 = _sc_conv(xs2, eidx4, zH)
    _, sp2 = _tc_self(h2, dp, ewp2, ebp2, wb2, sel)
    out_pad = _tc_postf(agg2, dp, h2, ewp2, ebp2, wa2, sel, sp2, wo, bo)
    return out_pad[:, :C]
